# baseline (speedup 1.0000x reference)
import functools

import jax
import jax.numpy as jnp
from jax import lax
from jax.experimental import pallas as pl
from jax.experimental.pallas import tpu as pltpu
from jax.experimental.pallas import tpu_sc as plsc

NCORES = 2
NSUB = 16
CH = 128
ACC_CORE = 0


def _sc_pass(x, packed3, n, d, cpt, spad, shift):
    zpt = spad // NSUB
    nz_full = zpt // CH
    zrem = zpt - nz_full * CH

    mesh = plsc.VectorSubcoreMesh(core_axis_name="c", subcore_axis_name="s")

    @functools.partial(
        pl.kernel,
        out_type=[
            jax.ShapeDtypeStruct((spad, d), jnp.float32),
            jax.ShapeDtypeStruct((spad, d), jnp.float32),
        ],
        mesh=mesh,
        scratch_types=[
            pltpu.VMEM((cpt // 2, CH), jnp.int32),
            pltpu.VMEM((CH,), jnp.int32),
            pltpu.VMEM((CH,), jnp.int32),
            pltpu.VMEM((CH,), jnp.int32),
            pltpu.VMEM((CH,), jnp.int32),
            pltpu.VMEM((CH, d), jnp.float32),
            pltpu.VMEM((CH, d), jnp.float32),
            pltpu.VMEM_SHARED((spad, d), jnp.float32),
            pltpu.SemaphoreType.DMA,
            pltpu.SemaphoreType.DMA,
        ],
    )
    def k(x_hbm, p_hbm, acc_out, deg_out,
          p_v, sA, dA, sB, dB, bufa, bufb, sh, sema, semb):
        cid = lax.axis_index("c")
        sid = lax.axis_index("s")

        def fill(i, carry):
            for cc in range(d // 16):
                bufa[i, pl.ds(cc * 16, 16)] = jnp.zeros((16,), jnp.float32)
            return carry
        lax.fori_loop(0, CH, fill, 0)

        zbase = sid * zpt
        for kk in range(nz_full):
            pltpu.sync_copy(bufa, sh.at[pl.ds(zbase + kk * CH, CH)])
        if zrem:
            pltpu.sync_copy(bufa.at[pl.ds(0, zrem)],
                            sh.at[pl.ds(zbase + nz_full * CH, zrem)])
        plsc.subcore_barrier()

        mask = (1 << shift) - 1

        def unpack(j, sref, dref):
            def qq(q, c):
                pv = p_v[j, pl.ds(q * 16, 16)]
                sref[pl.ds(q * 16, 16)] = lax.shift_right_logical(pv, shift)
                dref[pl.ds(q * 16, 16)] = jnp.bitwise_and(pv, mask)
                return c
            lax.fori_loop(0, CH // 16, qq, 0)

        hpt = cpt // 2

        @pl.when(cid == ACC_CORE)
        def _():
            for h in range(2):
                pltpu.sync_copy(p_hbm.at[sid, pl.ds(h * hpt, hpt)], p_v)
                unpack(0, sA, dA)
                pltpu.async_copy(x_hbm.at[sA], bufa, sema)
                unpack(1, sB, dB)
                pltpu.async_copy(x_hbm.at[sB], bufb, semb)

                def pair(t, carry):
                    j0 = 2 * t
                    pltpu.make_async_copy(x_hbm.at[sA], bufa, sema).wait()
                    pltpu.sync_copy(bufa, sh.at[dA], add=True)
                    @pl.when(j0 + 2 < hpt)
                    def _():
                        unpack(j0 + 2, sA, dA)
                        pltpu.async_copy(x_hbm.at[sA], bufa, sema)
                    pltpu.make_async_copy(x_hbm.at[sB], bufb, semb).wait()
                    pltpu.sync_copy(bufb, sh.at[dB], add=True)
                    @pl.when(j0 + 3 < hpt)
                    def _():
                        unpack(j0 + 3, sB, dB)
                        pltpu.async_copy(x_hbm.at[sB], bufb, semb)
                    return carry
                lax.fori_loop(0, hpt // 2, pair, 0)

        @pl.when(cid != ACC_CORE)
        def _():
            def fillb(i, carry):
                for cc in range(d // 16):
                    bufb[i, pl.ds(cc * 16, 16)] = jnp.zeros((16,), jnp.float32)
                bufb[i, pl.ds(0, 16)] = jnp.ones((16,), jnp.float32)
                return carry
            lax.fori_loop(0, CH, fillb, 0)

            def chunk(j, carry):
                def qq(q, c):
                    pv = p_v[j, pl.ds(q * 16, 16)]
                    dA[pl.ds(q * 16, 16)] = jnp.bitwise_and(pv, mask)
                    return c
                lax.fori_loop(0, CH // 16, qq, 0)
                pltpu.sync_copy(bufb, sh.at[dA], add=True)
                return carry

            for h in range(2):
                pltpu.sync_copy(p_hbm.at[sid, pl.ds(h * hpt, hpt)], p_v)
                lax.fori_loop(0, hpt, chunk, 0)

        plsc.subcore_barrier()

        @pl.when(cid == ACC_CORE)
        def _():
            pltpu.sync_copy(sh.at[pl.ds(zbase, zpt)],
                            acc_out.at[pl.ds(zbase, zpt)])

        @pl.when(cid != ACC_CORE)
        def _():
            pltpu.sync_copy(sh.at[pl.ds(zbase, zpt)],
                            deg_out.at[pl.ds(zbase, zpt)])

    return k(x, packed3)


def _combine(acc, deg, x, w_self, w_nbr, b2, n, d):
    r = 1000 if n % 1000 == 0 else n

    def body(acc_ref, deg_ref, x_ref, ws_ref, wn_ref, b_ref, o_ref):
        dg = jnp.maximum(deg_ref[:, 0], 1.0)
        a = acc_ref[...] / dg[:, None]
        agg = jnp.dot(a, wn_ref[...], preferred_element_type=jnp.float32)
        self_t = jnp.dot(x_ref[...], ws_ref[...], preferred_element_type=jnp.float32)
        o_ref[...] = jnp.maximum(agg + self_t + b_ref[...], 0.0)

    return pl.pallas_call(
        body,
        grid=(n // r,),
        in_specs=[
            pl.BlockSpec((r, d), lambda i: (i, 0)),
            pl.BlockSpec((r, d), lambda i: (i, 0)),
            pl.BlockSpec((r, d), lambda i: (i, 0)),
            pl.BlockSpec((d, d), lambda i: (0, 0)),
            pl.BlockSpec((d, d), lambda i: (0, 0)),
            pl.BlockSpec((1, d), lambda i: (0, 0)),
        ],
        out_specs=pl.BlockSpec((r, d), lambda i: (i, 0)),
        out_shape=jax.ShapeDtypeStruct((n, d), jnp.float32),
    )(acc, deg, x, w_self, w_nbr, b2)


def kernel(x, edge_index, W_self, W_nbr, b):
    n, d = x.shape
    e = edge_index.shape[1]
    cpt = 16 * (-(-e // (NSUB * CH * 16)))
    epad = NSUB * CH * cpt
    spad = -(-(n + 1) // (NSUB * 8)) * (NSUB * 8)
    shift = max(int(n).bit_length(), 1)
    assert (n - 1) < (1 << (31 - shift)), "node count too large for packing"
    src = edge_index[0]
    dst = edge_index[1]
    if epad != e:
        pad = epad - e
        src = jnp.concatenate([src, jnp.zeros((pad,), jnp.int32)])
        dst = jnp.concatenate([dst, jnp.full((pad,), n, jnp.int32)])
    packed3 = (jnp.left_shift(src, shift) | dst).reshape(NSUB, cpt, CH)
    acc, deg = _sc_pass(x, packed3, n, d, cpt, spad, shift)
    return _combine(acc, deg, x, W_self, W_nbr, b.reshape(1, d), n, d)


# speedup vs baseline: 1.0574x; 1.0574x over previous
import functools

import jax
import jax.numpy as jnp
from jax import lax
from jax.experimental import pallas as pl
from jax.experimental.pallas import tpu as pltpu
from jax.experimental.pallas import tpu_sc as plsc

NCORES = 2
NSUB = 16
CH = 128
ACC_CORE = 1


def _sc_pass(x, packed3, n, d, cpt, spad, shift):
    zpt = spad // NSUB
    nz_full = zpt // CH
    zrem = zpt - nz_full * CH

    mesh = plsc.VectorSubcoreMesh(core_axis_name="c", subcore_axis_name="s")

    @functools.partial(
        pl.kernel,
        out_type=[
            jax.ShapeDtypeStruct((spad, d), jnp.float32),
            jax.ShapeDtypeStruct((spad, d), jnp.float32),
        ],
        mesh=mesh,
        scratch_types=[
            pltpu.VMEM((cpt // 2, CH), jnp.int32),
            pltpu.VMEM((CH,), jnp.int32),
            pltpu.VMEM((CH,), jnp.int32),
            pltpu.VMEM((CH,), jnp.int32),
            pltpu.VMEM((CH,), jnp.int32),
            pltpu.VMEM((CH, d), jnp.float32),
            pltpu.VMEM((CH, d), jnp.float32),
            pltpu.VMEM_SHARED((spad, d), jnp.float32),
            pltpu.SemaphoreType.DMA,
            pltpu.SemaphoreType.DMA,
        ],
    )
    def k(x_hbm, p_hbm, acc_out, deg_out,
          p_v, sA, dA, sB, dB, bufa, bufb, sh, sema, semb):
        cid = lax.axis_index("c")
        sid = lax.axis_index("s")

        def fill(i, carry):
            for cc in range(d // 16):
                bufa[i, pl.ds(cc * 16, 16)] = jnp.zeros((16,), jnp.float32)
            return carry
        lax.fori_loop(0, CH, fill, 0)

        zbase = sid * zpt
        for kk in range(nz_full):
            pltpu.sync_copy(bufa, sh.at[pl.ds(zbase + kk * CH, CH)])
        if zrem:
            pltpu.sync_copy(bufa.at[pl.ds(0, zrem)],
                            sh.at[pl.ds(zbase + nz_full * CH, zrem)])
        plsc.subcore_barrier()

        mask = (1 << shift) - 1

        def unpack(j, sref, dref):
            def qq(q, c):
                pv = p_v[j, pl.ds(q * 16, 16)]
                sref[pl.ds(q * 16, 16)] = lax.shift_right_logical(pv, shift)
                dref[pl.ds(q * 16, 16)] = jnp.bitwise_and(pv, mask)
                return c
            lax.fori_loop(0, CH // 16, qq, 0)

        hpt = cpt // 2

        @pl.when(cid == ACC_CORE)
        def _():
            for h in range(2):
                pltpu.sync_copy(p_hbm.at[sid, pl.ds(h * hpt, hpt)], p_v)
                unpack(0, sA, dA)
                pltpu.async_copy(x_hbm.at[sA], bufa, sema)
                unpack(1, sB, dB)
                pltpu.async_copy(x_hbm.at[sB], bufb, semb)

                def pair(t, carry):
                    j0 = 2 * t
                    pltpu.make_async_copy(x_hbm.at[sA], bufa, sema).wait()
                    pltpu.sync_copy(bufa, sh.at[dA], add=True)
                    @pl.when(j0 + 2 < hpt)
                    def _():
                        unpack(j0 + 2, sA, dA)
                        pltpu.async_copy(x_hbm.at[sA], bufa, sema)
                    pltpu.make_async_copy(x_hbm.at[sB], bufb, semb).wait()
                    pltpu.sync_copy(bufb, sh.at[dB], add=True)
                    @pl.when(j0 + 3 < hpt)
                    def _():
                        unpack(j0 + 3, sB, dB)
                        pltpu.async_copy(x_hbm.at[sB], bufb, semb)
                    return carry
                lax.fori_loop(0, hpt // 2, pair, 0)

        @pl.when(cid != ACC_CORE)
        def _():
            def fillb(i, carry):
                for cc in range(d // 16):
                    bufb[i, pl.ds(cc * 16, 16)] = jnp.zeros((16,), jnp.float32)
                bufb[i, pl.ds(0, 16)] = jnp.ones((16,), jnp.float32)
                return carry
            lax.fori_loop(0, CH, fillb, 0)

            def chunk(j, carry):
                def qq(q, c):
                    pv = p_v[j, pl.ds(q * 16, 16)]
                    dA[pl.ds(q * 16, 16)] = jnp.bitwise_and(pv, mask)
                    return c
                lax.fori_loop(0, CH // 16, qq, 0)
                pltpu.sync_copy(bufb, sh.at[dA], add=True)
                return carry

            for h in range(2):
                pltpu.sync_copy(p_hbm.at[sid, pl.ds(h * hpt, hpt)], p_v)
                lax.fori_loop(0, hpt, chunk, 0)

        plsc.subcore_barrier()

        @pl.when(cid == ACC_CORE)
        def _():
            pltpu.sync_copy(sh.at[pl.ds(zbase, zpt)],
                            acc_out.at[pl.ds(zbase, zpt)])

        @pl.when(cid != ACC_CORE)
        def _():
            pltpu.sync_copy(sh.at[pl.ds(zbase, zpt)],
                            deg_out.at[pl.ds(zbase, zpt)])

    return k(x, packed3)


def _combine(acc, deg, x, w_self, w_nbr, b2, n, d):
    r = 1000 if n % 1000 == 0 else n

    def body(acc_ref, deg_ref, x_ref, ws_ref, wn_ref, b_ref, o_ref):
        dg = jnp.maximum(deg_ref[:, 0], 1.0)
        a = acc_ref[...] / dg[:, None]
        agg = jnp.dot(a, wn_ref[...], preferred_element_type=jnp.float32)
        self_t = jnp.dot(x_ref[...], ws_ref[...], preferred_element_type=jnp.float32)
        o_ref[...] = jnp.maximum(agg + self_t + b_ref[...], 0.0)

    return pl.pallas_call(
        body,
        grid=(n // r,),
        in_specs=[
            pl.BlockSpec((r, d), lambda i: (i, 0)),
            pl.BlockSpec((r, d), lambda i: (i, 0)),
            pl.BlockSpec((r, d), lambda i: (i, 0)),
            pl.BlockSpec((d, d), lambda i: (0, 0)),
            pl.BlockSpec((d, d), lambda i: (0, 0)),
            pl.BlockSpec((1, d), lambda i: (0, 0)),
        ],
        out_specs=pl.BlockSpec((r, d), lambda i: (i, 0)),
        out_shape=jax.ShapeDtypeStruct((n, d), jnp.float32),
    )(acc, deg, x, w_self, w_nbr, b2)


def kernel(x, edge_index, W_self, W_nbr, b):
    n, d = x.shape
    e = edge_index.shape[1]
    cpt = 16 * (-(-e // (NSUB * CH * 16)))
    epad = NSUB * CH * cpt
    spad = -(-(n + 1) // (NSUB * 8)) * (NSUB * 8)
    shift = max(int(n).bit_length(), 1)
    assert (n - 1) < (1 << (31 - shift)), "node count too large for packing"
    src = edge_index[0]
    dst = edge_index[1]
    if epad != e:
        pad = epad - e
        src = jnp.concatenate([src, jnp.zeros((pad,), jnp.int32)])
        dst = jnp.concatenate([dst, jnp.full((pad,), n, jnp.int32)])
    packed3 = (jnp.left_shift(src, shift) | dst).reshape(NSUB, cpt, CH)
    acc, deg = _sc_pass(x, packed3, n, d, cpt, spad, shift)
    return _combine(acc, deg, x, W_self, W_nbr, b.reshape(1, d), n, d)


# speedup vs baseline: 1.1926x; 1.1279x over previous
import functools

import jax
import jax.numpy as jnp
from jax import lax
from jax.experimental import pallas as pl
from jax.experimental.pallas import tpu as pltpu
from jax.experimental.pallas import tpu_sc as plsc

NCORES = 2
NSUB = 16
NW = NCORES * NSUB
CH = 128


def _sc_acc(x, src1, dst1, n, d, cpt, spad):
    zpt = spad // NSUB
    nz_full = zpt // CH
    zrem = zpt - nz_full * CH

    mesh = plsc.VectorSubcoreMesh(core_axis_name="c", subcore_axis_name="s")

    @functools.partial(
        pl.kernel,
        out_type=jax.ShapeDtypeStruct((NCORES, spad, d), jnp.float32),
        mesh=mesh,
        scratch_types=[
            pltpu.VMEM((CH,), jnp.int32),
            pltpu.VMEM((CH,), jnp.int32),
            pltpu.VMEM((CH, d), jnp.float32),
            pltpu.VMEM_SHARED((spad, d), jnp.float32),
            pltpu.SemaphoreType.DMA,
        ],
    )
    def k(x_hbm, src_hbm, dst_hbm, acc_out,
          src_v, dst_v, rows_v, acc_sh, sem):
        cid = lax.axis_index("c")
        sid = lax.axis_index("s")
        wid = cid * NSUB + sid

        def fill(i, carry):
            for cc in range(d // 16):
                rows_v[i, pl.ds(cc * 16, 16)] = jnp.zeros((16,), jnp.float32)
            return carry
        lax.fori_loop(0, CH, fill, 0)

        zbase = sid * zpt
        for kk in range(nz_full):
            pltpu.sync_copy(rows_v, acc_sh.at[pl.ds(zbase + kk * CH, CH)])
        if zrem:
            pltpu.sync_copy(rows_v.at[pl.ds(0, zrem)],
                            acc_sh.at[pl.ds(zbase + nz_full * CH, zrem)])
        plsc.subcore_barrier()

        tbase = wid * (cpt * CH)
        def chunk(j, carry):
            pltpu.sync_copy(src_hbm.at[pl.ds(tbase + j * CH, CH)], src_v)
            pltpu.sync_copy(dst_hbm.at[pl.ds(tbase + j * CH, CH)], dst_v)
            pltpu.async_copy(x_hbm.at[src_v], rows_v, sem).wait()
            pltpu.sync_copy(rows_v, acc_sh.at[dst_v], add=True)
            return carry
        lax.fori_loop(0, cpt, chunk, 0)
        plsc.subcore_barrier()

        pltpu.sync_copy(acc_sh.at[pl.ds(zbase, zpt)],
                        acc_out.at[cid, pl.ds(zbase, zpt)])

    return k(x, src1, dst1)


def _sc_deg(dst1, n, d, cpt, spad):
    zpt = spad // NSUB
    nz_full = zpt // CH
    zrem = zpt - nz_full * CH

    mesh = plsc.VectorSubcoreMesh(core_axis_name="c", subcore_axis_name="s")

    @functools.partial(
        pl.kernel,
        out_type=jax.ShapeDtypeStruct((NCORES, spad, d), jnp.float32),
        mesh=mesh,
        scratch_types=[
            pltpu.VMEM((CH,), jnp.int32),
            pltpu.VMEM((CH, d), jnp.float32),
            pltpu.VMEM_SHARED((spad, d), jnp.float32),
        ],
    )
    def k(dst_hbm, deg_out, dst_v, ones_v, deg_sh):
        cid = lax.axis_index("c")
        sid = lax.axis_index("s")
        wid = cid * NSUB + sid

        def fill(i, carry):
            for cc in range(d // 16):
                ones_v[i, pl.ds(cc * 16, 16)] = jnp.zeros((16,), jnp.float32)
            return carry
        lax.fori_loop(0, CH, fill, 0)

        zbase = sid * zpt
        for kk in range(nz_full):
            pltpu.sync_copy(ones_v, deg_sh.at[pl.ds(zbase + kk * CH, CH)])
        if zrem:
            pltpu.sync_copy(ones_v.at[pl.ds(0, zrem)],
                            deg_sh.at[pl.ds(zbase + nz_full * CH, zrem)])

        def fill1(i, carry):
            ones_v[i, pl.ds(0, 16)] = jnp.ones((16,), jnp.float32)
            return carry
        lax.fori_loop(0, CH, fill1, 0)
        plsc.subcore_barrier()

        tbase = wid * (cpt * CH)
        def chunk(j, carry):
            pltpu.sync_copy(dst_hbm.at[pl.ds(tbase + j * CH, CH)], dst_v)
            pltpu.sync_copy(ones_v, deg_sh.at[dst_v], add=True)
            return carry
        lax.fori_loop(0, cpt, chunk, 0)
        plsc.subcore_barrier()

        pltpu.sync_copy(deg_sh.at[pl.ds(zbase, zpt)],
                        deg_out.at[cid, pl.ds(zbase, zpt)])

    return k(dst1)


def _combine(acc, deg, x, w_self, w_nbr, b2, n, d):
    r = 1000 if n % 1000 == 0 else n

    def body(acc_ref, deg_ref, x_ref, ws_ref, wn_ref, b_ref, o_ref):
        a = acc_ref[0] + acc_ref[1]
        dg = deg_ref[0, :, 0] + deg_ref[1, :, 0]
        dg = jnp.maximum(dg, 1.0)
        a = a / dg[:, None]
        agg = jnp.dot(a, wn_ref[...], preferred_element_type=jnp.float32)
        self_t = jnp.dot(x_ref[...], ws_ref[...], preferred_element_type=jnp.float32)
        o_ref[...] = jnp.maximum(agg + self_t + b_ref[...], 0.0)

    return pl.pallas_call(
        body,
        grid=(n // r,),
        in_specs=[
            pl.BlockSpec((2, r, d), lambda i: (0, i, 0)),
            pl.BlockSpec((2, r, d), lambda i: (0, i, 0)),
            pl.BlockSpec((r, d), lambda i: (i, 0)),
            pl.BlockSpec((d, d), lambda i: (0, 0)),
            pl.BlockSpec((d, d), lambda i: (0, 0)),
            pl.BlockSpec((1, d), lambda i: (0, 0)),
        ],
        out_specs=pl.BlockSpec((r, d), lambda i: (i, 0)),
        out_shape=jax.ShapeDtypeStruct((n, d), jnp.float32),
    )(acc, deg, x, w_self, w_nbr, b2)


def kernel(x, edge_index, W_self, W_nbr, b):
    n, d = x.shape
    e = edge_index.shape[1]
    cpt = -(-e // (NW * CH))
    epad = NW * CH * cpt
    spad = -(-(n + 1) // (NSUB * 8)) * (NSUB * 8)
    src = edge_index[0]
    dst = edge_index[1]
    if epad != e:
        pad = epad - e
        src = jnp.concatenate([src, jnp.zeros((pad,), jnp.int32)])
        dst = jnp.concatenate([dst, jnp.full((pad,), n, jnp.int32)])
    acc = _sc_acc(x, src, dst, n, d, cpt, spad)
    deg = _sc_deg(dst, n, d, cpt, spad)
    return _combine(acc, deg, x, W_self, W_nbr, b.reshape(1, d), n, d)
